# baseline (device time: 49535 ns/iter reference)
import jax
import jax.numpy as jnp
from jax import lax
from jax.experimental import pallas as pl
from jax.experimental.pallas import tpu as pltpu


def kernel(dy, W):
    m, f = dy.shape
    d = W.shape[0]

    def body(dy_ref, w_ref, out_ref, send_ref, recv_ref, send_sem, recv_sem):
        my_x = lax.axis_index("x")
        my_y = lax.axis_index("y")
        my_z = lax.axis_index("z")
        peer = (1 - my_x, my_y, my_z)

        barrier = pltpu.get_barrier_semaphore()
        pl.semaphore_signal(
            barrier, inc=1, device_id=peer, device_id_type=pl.DeviceIdType.MESH
        )
        pl.semaphore_wait(barrier, 1)

        partial = lax.dot_general(
            dy_ref[...].astype(jnp.bfloat16),
            w_ref[...].astype(jnp.bfloat16),
            dimension_numbers=(((1,), (1,)), ((), ())),
            preferred_element_type=jnp.float32,
        )
        send_ref[...] = partial.astype(jnp.bfloat16)
        rdma = pltpu.make_async_remote_copy(
            src_ref=send_ref,
            dst_ref=recv_ref,
            send_sem=send_sem,
            recv_sem=recv_sem,
            device_id=peer,
            device_id_type=pl.DeviceIdType.MESH,
        )
        rdma.start()
        rdma.wait()
        out_ref[...] = partial + recv_ref[...].astype(jnp.float32)

    return pl.pallas_call(
        body,
        out_shape=jax.ShapeDtypeStruct((m, d), jnp.float32),
        in_specs=[
            pl.BlockSpec(memory_space=pltpu.VMEM),
            pl.BlockSpec(memory_space=pltpu.VMEM),
        ],
        out_specs=pl.BlockSpec(memory_space=pltpu.VMEM),
        scratch_shapes=[
            pltpu.VMEM((m, d), jnp.bfloat16),
            pltpu.VMEM((m, d), jnp.bfloat16),
            pltpu.SemaphoreType.DMA,
            pltpu.SemaphoreType.DMA,
        ],
        compiler_params=pltpu.CompilerParams(collective_id=0),
    )(dy, W)


# device time: 42457 ns/iter; 1.1667x vs baseline; 1.1667x over previous
import jax
import jax.numpy as jnp
from jax import lax
from jax.experimental import pallas as pl
from jax.experimental.pallas import tpu as pltpu

K = 4


def kernel(dy, W):
    m, f = dy.shape
    d = W.shape[0]
    mc = m // K

    def body(dy_ref, w_ref, out_ref, send_ref, recv_ref, send_sems, recv_sems):
        my_x = lax.axis_index("x")
        my_y = lax.axis_index("y")
        my_z = lax.axis_index("z")
        peer = (1 - my_x, my_y, my_z)

        barrier = pltpu.get_barrier_semaphore()
        pl.semaphore_signal(
            barrier, inc=1, device_id=peer, device_id_type=pl.DeviceIdType.MESH
        )
        pl.semaphore_wait(barrier, 1)

        w_bf16 = w_ref[...].astype(jnp.bfloat16)

        def chunk_rdma(k):
            rows = pl.ds(k * mc, mc)
            return pltpu.make_async_remote_copy(
                src_ref=send_ref.at[rows, :],
                dst_ref=recv_ref.at[rows, :],
                send_sem=send_sems.at[k],
                recv_sem=recv_sems.at[k],
                device_id=peer,
                device_id_type=pl.DeviceIdType.MESH,
            )

        for k in range(K):
            rows = pl.ds(k * mc, mc)
            partial = lax.dot_general(
                dy_ref[rows, :].astype(jnp.bfloat16),
                w_bf16,
                dimension_numbers=(((1,), (1,)), ((), ())),
                preferred_element_type=jnp.float32,
            )
            out_ref[rows, :] = partial
            send_ref[rows, :] = partial.astype(jnp.bfloat16)
            chunk_rdma(k).start()

        for k in range(K):
            rows = pl.ds(k * mc, mc)
            r = chunk_rdma(k)
            r.wait_recv()
            out_ref[rows, :] = out_ref[rows, :] + recv_ref[rows, :].astype(
                jnp.float32
            )
            r.wait_send()

    return pl.pallas_call(
        body,
        out_shape=jax.ShapeDtypeStruct((m, d), jnp.float32),
        in_specs=[
            pl.BlockSpec(memory_space=pltpu.VMEM),
            pl.BlockSpec(memory_space=pltpu.VMEM),
        ],
        out_specs=pl.BlockSpec(memory_space=pltpu.VMEM),
        scratch_shapes=[
            pltpu.VMEM((m, d), jnp.bfloat16),
            pltpu.VMEM((m, d), jnp.bfloat16),
            pltpu.SemaphoreType.DMA((K,)),
            pltpu.SemaphoreType.DMA((K,)),
        ],
        compiler_params=pltpu.CompilerParams(collective_id=0),
    )(dy, W)


# device time: 39284 ns/iter; 1.2609x vs baseline; 1.0808x over previous
import jax
import jax.numpy as jnp
from jax import lax
from jax.experimental import pallas as pl
from jax.experimental.pallas import tpu as pltpu

MQ = 256


def kernel(dy, W):
    m, f = dy.shape
    d = W.shape[0]

    def body(
        dy_hbm,
        w_hbm,
        out_ref,
        w_f32,
        w_bf,
        dy_f32,
        send_bf,
        recv_bf,
        w_sem,
        dy_sems,
        x_send_sems,
        x_recv_sems,
        fwd_send_sems,
        fwd_recv_sems,
    ):
        my_x = lax.axis_index("x")
        my_y = lax.axis_index("y")
        my_z = lax.axis_index("z")
        py = lax.rem(my_y, 2)
        pz = lax.rem(my_z, 2)
        x_peer = (1 - my_x, my_y, my_z)
        y_partner = (my_x, my_y + 1 - 2 * py, my_z)
        z_partner = (my_x, my_y, my_z + 1 - 2 * pz)

        jA = 2 * py + pz
        jD = 3 - jA
        jB = 2 * (1 - py) + pz
        jC = 2 * py + (1 - pz)

        def q(j):
            return pl.ds(j * MQ, MQ)

        barrier = pltpu.get_barrier_semaphore()
        for nbr in (x_peer, y_partner, z_partner):
            pl.semaphore_signal(
                barrier, inc=1, device_id=nbr,
                device_id_type=pl.DeviceIdType.MESH,
            )
        pl.semaphore_wait(barrier, 3)

        w_cp = pltpu.make_async_copy(w_hbm, w_f32, w_sem)
        w_cp.start()
        order = (jA, jD, jB, jC)
        dy_cps = []
        for t, j in enumerate(order):
            cp = pltpu.make_async_copy(
                dy_hbm.at[q(j), :], dy_f32.at[q(j), :], dy_sems.at[t]
            )
            cp.start()
            dy_cps.append(cp)

        w_cp.wait()
        w_bf[...] = w_f32[...].astype(jnp.bfloat16)

        def x_rdma(t, j):
            return pltpu.make_async_remote_copy(
                src_ref=send_bf.at[q(j), :],
                dst_ref=recv_bf.at[q(j), :],
                send_sem=x_send_sems.at[t],
                recv_sem=x_recv_sems.at[t],
                device_id=x_peer,
                device_id_type=pl.DeviceIdType.MESH,
            )

        for t, j in enumerate(order):
            dy_cps[t].wait()
            partial = lax.dot_general(
                dy_f32[q(j), :].astype(jnp.bfloat16),
                w_bf[...],
                dimension_numbers=(((1,), (1,)), ((), ())),
                preferred_element_type=jnp.float32,
            )
            out_ref[q(j), :] = partial
            if t < 2:
                send_bf[q(j), :] = partial.astype(jnp.bfloat16)
                x_rdma(t, j).start()

        x0 = x_rdma(0, jA)
        x0.wait_recv()
        fwds = []
        for i, partner in enumerate((y_partner, z_partner)):
            fwd = pltpu.make_async_remote_copy(
                src_ref=recv_bf.at[q(jA), :],
                dst_ref=recv_bf.at[q(jA), :],
                send_sem=fwd_send_sems.at[i],
                recv_sem=fwd_recv_sems.at[i],
                device_id=partner,
                device_id_type=pl.DeviceIdType.MESH,
            )
            fwd.start()
            fwds.append(fwd)

        x1 = x_rdma(1, jD)
        x1.wait_recv()
        for fwd in fwds:
            fwd.wait_recv()

        out_ref[...] = out_ref[...] + recv_bf[...].astype(jnp.float32)

        x0.wait_send()
        x1.wait_send()
        for fwd in fwds:
            fwd.wait_send()

    return pl.pallas_call(
        body,
        out_shape=jax.ShapeDtypeStruct((m, d), jnp.float32),
        in_specs=[
            pl.BlockSpec(memory_space=pl.ANY),
            pl.BlockSpec(memory_space=pl.ANY),
        ],
        out_specs=pl.BlockSpec(memory_space=pltpu.VMEM),
        scratch_shapes=[
            pltpu.VMEM((d, f), jnp.float32),
            pltpu.VMEM((d, f), jnp.bfloat16),
            pltpu.VMEM((m, f), jnp.float32),
            pltpu.VMEM((m, d), jnp.bfloat16),
            pltpu.VMEM((m, d), jnp.bfloat16),
            pltpu.SemaphoreType.DMA,
            pltpu.SemaphoreType.DMA((4,)),
            pltpu.SemaphoreType.DMA((2,)),
            pltpu.SemaphoreType.DMA((2,)),
            pltpu.SemaphoreType.DMA((2,)),
            pltpu.SemaphoreType.DMA((2,)),
        ],
        compiler_params=pltpu.CompilerParams(
            collective_id=0, vmem_limit_bytes=100 * 1024 * 1024
        ),
    )(dy, W)


# device time: 37994 ns/iter; 1.3038x vs baseline; 1.0340x over previous
import jax
import jax.numpy as jnp
from jax import lax
from jax.experimental import pallas as pl
from jax.experimental.pallas import tpu as pltpu

MQ = 256


def kernel(dy, W):
    m, f = dy.shape
    d = W.shape[0]

    def body(
        dy_hbm,
        w_hbm,
        out_ref,
        w_f32,
        w_bf,
        dy_f32,
        send_bf,
        recv_bf,
        w_sem,
        dy_sems,
        x_send_sems,
        x_recv_sems,
        fwd_send_sems,
        fwd_recv_sems,
    ):
        my_x = lax.axis_index("x")
        my_y = lax.axis_index("y")
        my_z = lax.axis_index("z")
        py = lax.rem(my_y, 2)
        pz = lax.rem(my_z, 2)
        x_peer = (1 - my_x, my_y, my_z)
        y_partner = (my_x, my_y + 1 - 2 * py, my_z)
        z_partner = (my_x, my_y, my_z + 1 - 2 * pz)

        jA = 2 * py + pz
        jD = 3 - jA
        jB = 2 * (1 - py) + pz
        jC = 2 * py + (1 - pz)

        def q(j):
            return pl.ds(j * MQ, MQ)

        barrier = pltpu.get_barrier_semaphore()
        for nbr in (x_peer, y_partner, z_partner):
            pl.semaphore_signal(
                barrier, inc=1, device_id=nbr,
                device_id_type=pl.DeviceIdType.MESH,
            )

        w_cp = pltpu.make_async_copy(w_hbm, w_f32, w_sem)
        w_cp.start()
        order = (jA, jD, jB, jC)
        dy_cps = []
        for t, j in enumerate(order):
            cp = pltpu.make_async_copy(
                dy_hbm.at[q(j), :], dy_f32.at[q(j), :], dy_sems.at[t]
            )
            cp.start()
            dy_cps.append(cp)

        w_cp.wait()
        w_bf[...] = w_f32[...].astype(jnp.bfloat16)

        def x_rdma(t, j):
            return pltpu.make_async_remote_copy(
                src_ref=send_bf.at[q(j), :],
                dst_ref=recv_bf.at[q(j), :],
                send_sem=x_send_sems.at[t],
                recv_sem=x_recv_sems.at[t],
                device_id=x_peer,
                device_id_type=pl.DeviceIdType.MESH,
            )

        for t, j in enumerate(order):
            dy_cps[t].wait()
            partial = lax.dot_general(
                dy_f32[q(j), :].astype(jnp.bfloat16),
                w_bf[...],
                dimension_numbers=(((1,), (1,)), ((), ())),
                preferred_element_type=jnp.float32,
            )
            out_ref[q(j), :] = partial
            if t < 2:
                send_bf[q(j), :] = partial.astype(jnp.bfloat16)
                if t == 0:
                    pl.semaphore_wait(barrier, 3)
                x_rdma(t, j).start()

        x0 = x_rdma(0, jA)
        x0.wait_recv()
        fwds = []
        for i, partner in enumerate((y_partner, z_partner)):
            fwd = pltpu.make_async_remote_copy(
                src_ref=recv_bf.at[q(jA), :],
                dst_ref=recv_bf.at[q(jA), :],
                send_sem=fwd_send_sems.at[i],
                recv_sem=fwd_recv_sems.at[i],
                device_id=partner,
                device_id_type=pl.DeviceIdType.MESH,
            )
            fwd.start()
            fwds.append(fwd)

        x1 = x_rdma(1, jD)
        x1.wait_recv()
        for fwd in fwds:
            fwd.wait_recv()

        out_ref[...] = out_ref[...] + recv_bf[...].astype(jnp.float32)

        x0.wait_send()
        x1.wait_send()
        for fwd in fwds:
            fwd.wait_send()

    return pl.pallas_call(
        body,
        out_shape=jax.ShapeDtypeStruct((m, d), jnp.float32),
        in_specs=[
            pl.BlockSpec(memory_space=pl.ANY),
            pl.BlockSpec(memory_space=pl.ANY),
        ],
        out_specs=pl.BlockSpec(memory_space=pltpu.VMEM),
        scratch_shapes=[
            pltpu.VMEM((d, f), jnp.float32),
            pltpu.VMEM((d, f), jnp.bfloat16),
            pltpu.VMEM((m, f), jnp.float32),
            pltpu.VMEM((m, d), jnp.bfloat16),
            pltpu.VMEM((m, d), jnp.bfloat16),
            pltpu.SemaphoreType.DMA,
            pltpu.SemaphoreType.DMA((4,)),
            pltpu.SemaphoreType.DMA((2,)),
            pltpu.SemaphoreType.DMA((2,)),
            pltpu.SemaphoreType.DMA((2,)),
            pltpu.SemaphoreType.DMA((2,)),
        ],
        compiler_params=pltpu.CompilerParams(
            collective_id=0, vmem_limit_bytes=100 * 1024 * 1024
        ),
    )(dy, W)


# device time: 37713 ns/iter; 1.3135x vs baseline; 1.0075x over previous
import jax
import jax.numpy as jnp
from jax import lax
from jax.experimental import pallas as pl
from jax.experimental.pallas import tpu as pltpu

MQ = 256


def kernel(dy, W):
    m, f = dy.shape
    d = W.shape[0]

    def body(
        dy_hbm,
        w_hbm,
        out_ref,
        w_f32,
        w_bf,
        dy_f32,
        send_bf,
        recv_bf,
        w_sem,
        dy_sems,
        x_send_sems,
        x_recv_sems,
        fwd_send_sems,
        fwd_recv_sems,
    ):
        my_x = lax.axis_index("x")
        my_y = lax.axis_index("y")
        my_z = lax.axis_index("z")
        py = lax.rem(my_y, 2)
        pz = lax.rem(my_z, 2)
        x_peer = (1 - my_x, my_y, my_z)
        y_partner = (my_x, my_y + 1 - 2 * py, my_z)
        z_partner = (my_x, my_y, my_z + 1 - 2 * pz)

        jA = 2 * py + pz
        jD = 3 - jA
        jB = 2 * (1 - py) + pz
        jC = 2 * py + (1 - pz)

        def q(j):
            return pl.ds(j * MQ, MQ)

        barrier = pltpu.get_barrier_semaphore()
        for nbr in (x_peer, y_partner, z_partner):
            pl.semaphore_signal(
                barrier, inc=1, device_id=nbr,
                device_id_type=pl.DeviceIdType.MESH,
            )

        w_cp = pltpu.make_async_copy(w_hbm, w_f32, w_sem)
        w_cp.start()
        order = (jA, jD, jB, jC)
        dy_cps = []
        for t, j in enumerate(order):
            cp = pltpu.make_async_copy(
                dy_hbm.at[q(j), :], dy_f32.at[q(j), :], dy_sems.at[t]
            )
            cp.start()
            dy_cps.append(cp)

        w_cp.wait()
        w_bf[...] = w_f32[...].astype(jnp.bfloat16)

        def x_rdma(t, j):
            return pltpu.make_async_remote_copy(
                src_ref=send_bf.at[q(j), :],
                dst_ref=recv_bf.at[q(j), :],
                send_sem=x_send_sems.at[t],
                recv_sem=x_recv_sems.at[t],
                device_id=x_peer,
                device_id_type=pl.DeviceIdType.MESH,
            )

        for t, j in enumerate(order):
            dy_cps[t].wait()
            partial = lax.dot_general(
                dy_f32[q(j), :].astype(jnp.bfloat16),
                w_bf[...],
                dimension_numbers=(((1,), (1,)), ((), ())),
                preferred_element_type=jnp.float32,
            )
            out_ref[q(j), :] = partial
            if t < 2:
                send_bf[q(j), :] = partial.astype(jnp.bfloat16)
                if t == 0:
                    pl.semaphore_wait(barrier, 3)
                x_rdma(t, j).start()

        x0 = x_rdma(0, jA)
        x0.wait_recv()
        fwds = []
        for i, partner in enumerate((y_partner, z_partner)):
            fwd = pltpu.make_async_remote_copy(
                src_ref=recv_bf.at[q(jA), :],
                dst_ref=recv_bf.at[q(jA), :],
                send_sem=fwd_send_sems.at[i],
                recv_sem=fwd_recv_sems.at[i],
                device_id=partner,
                device_id_type=pl.DeviceIdType.MESH,
            )
            fwd.start()
            fwds.append(fwd)

        x1 = x_rdma(1, jD)
        x1.wait_recv()
        for fwd in fwds:
            fwd.wait_recv()

        out_ref[...] = out_ref[...] + recv_bf[...].astype(jnp.float32)

        x0.wait_send()
        x1.wait_send()
        for fwd in fwds:
            fwd.wait_send()

    return pl.pallas_call(
        body,
        out_shape=jax.ShapeDtypeStruct((m, d), jnp.float32),
        in_specs=[
            pl.BlockSpec(memory_space=pl.ANY),
            pl.BlockSpec(memory_space=pl.ANY),
        ],
        out_specs=pl.BlockSpec(memory_space=pltpu.VMEM),
        scratch_shapes=[
            pltpu.VMEM((d, f), jnp.float32),
            pltpu.VMEM((d, f), jnp.bfloat16),
            pltpu.VMEM((m, f), jnp.float32),
            pltpu.VMEM((m, d), jnp.bfloat16),
            pltpu.VMEM((m, d), jnp.bfloat16),
            pltpu.SemaphoreType.DMA,
            pltpu.SemaphoreType.DMA((4,)),
            pltpu.SemaphoreType.DMA((2,)),
            pltpu.SemaphoreType.DMA((2,)),
            pltpu.SemaphoreType.DMA((2,)),
            pltpu.SemaphoreType.DMA((2,)),
        ],
        compiler_params=pltpu.CompilerParams(
            collective_id=0,
            vmem_limit_bytes=100 * 1024 * 1024,
            skip_device_barrier=True,
        ),
    )(dy, W)


# device time: 30604 ns/iter; 1.6186x vs baseline; 1.2323x over previous
import jax
import jax.numpy as jnp
from jax import lax
from jax.experimental import pallas as pl
from jax.experimental.pallas import tpu as pltpu

MQ = 256
DC = 256
NC = 4


def kernel(dy, W):
    m, f = dy.shape
    d = W.shape[0]

    def body(
        dy_hbm,
        w_hbm,
        out_ref,
        w_f32,
        w_bf,
        dy_f32,
        dy_bf,
        send_bf,
        recv_bf,
        w_sems,
        dy_sems,
        x_send_sems,
        x_recv_sems,
        fwd_send_sems,
        fwd_recv_sems,
    ):
        my_x = lax.axis_index("x")
        my_y = lax.axis_index("y")
        my_z = lax.axis_index("z")
        py = lax.rem(my_y, 2)
        pz = lax.rem(my_z, 2)
        x_peer = (1 - my_x, my_y, my_z)
        y_partner = (my_x, my_y + 1 - 2 * py, my_z)
        z_partner = (my_x, my_y, my_z + 1 - 2 * pz)

        jA = 2 * py + pz
        jD = 3 - jA
        order = (jA, jD, 2 * (1 - py) + pz, 2 * py + (1 - pz))

        def q(j):
            return pl.ds(j * MQ, MQ)

        def col(c):
            return pl.ds(c * DC, DC)

        barrier = pltpu.get_barrier_semaphore()
        for nbr in (x_peer, y_partner, z_partner):
            pl.semaphore_signal(
                barrier, inc=1, device_id=nbr,
                device_id_type=pl.DeviceIdType.MESH,
            )

        w_cps = [
            pltpu.make_async_copy(
                w_hbm.at[col(c), :], w_f32.at[col(c), :], w_sems.at[c]
            )
            for c in range(NC)
        ]
        dy_cps = [
            pltpu.make_async_copy(
                dy_hbm.at[q(j), :], dy_f32.at[q(j), :], dy_sems.at[t]
            )
            for t, j in enumerate(order)
        ]
        w_cps[0].start()
        dy_cps[0].start()
        for c in range(1, NC):
            w_cps[c].start()
        for t in range(1, 4):
            dy_cps[t].start()

        def x_piece(t, j, c):
            return pltpu.make_async_remote_copy(
                src_ref=send_bf.at[q(j), col(c)],
                dst_ref=recv_bf.at[q(j), col(c)],
                send_sem=x_send_sems.at[t * NC + c],
                recv_sem=x_recv_sems.at[t * NC + c],
                device_id=x_peer,
                device_id_type=pl.DeviceIdType.MESH,
            )

        def fwd_piece(i, partner, c):
            return pltpu.make_async_remote_copy(
                src_ref=recv_bf.at[q(jA), col(c)],
                dst_ref=recv_bf.at[q(jA), col(c)],
                send_sem=fwd_send_sems.at[i * NC + c],
                recv_sem=fwd_recv_sems.at[i * NC + c],
                device_id=partner,
                device_id_type=pl.DeviceIdType.MESH,
            )

        for t, j in enumerate(order):
            dy_cps[t].wait()
            dy_bf[...] = dy_f32[q(j), :].astype(jnp.bfloat16)
            for c in range(NC):
                if t == 0:
                    w_cps[c].wait()
                    w_bf[col(c), :] = w_f32[col(c), :].astype(jnp.bfloat16)
                piece = lax.dot_general(
                    dy_bf[...],
                    w_bf[col(c), :],
                    dimension_numbers=(((1,), (1,)), ((), ())),
                    preferred_element_type=jnp.float32,
                )
                out_ref[q(j), col(c)] = piece
                if t < 2:
                    send_bf[q(j), col(c)] = piece.astype(jnp.bfloat16)
                    if t == 0 and c == 0:
                        pl.semaphore_wait(barrier, 3)
                    x_piece(t, j, c).start()
                if t == 1:
                    x_piece(0, jA, c).wait_recv()
                    for i, partner in enumerate((y_partner, z_partner)):
                        fwd_piece(i, partner, c).start()

        for c in range(NC):
            x_piece(1, jD, c).wait_recv()
        for i, partner in enumerate((y_partner, z_partner)):
            for c in range(NC):
                fwd_piece(i, partner, c).wait_recv()

        out_ref[...] = out_ref[...] + recv_bf[...].astype(jnp.float32)

        for t in range(2):
            for c in range(NC):
                x_piece(t, order[t], c).wait_send()
        for i, partner in enumerate((y_partner, z_partner)):
            for c in range(NC):
                fwd_piece(i, partner, c).wait_send()

    return pl.pallas_call(
        body,
        out_shape=jax.ShapeDtypeStruct((m, d), jnp.float32),
        in_specs=[
            pl.BlockSpec(memory_space=pl.ANY),
            pl.BlockSpec(memory_space=pl.ANY),
        ],
        out_specs=pl.BlockSpec(memory_space=pltpu.VMEM),
        scratch_shapes=[
            pltpu.VMEM((d, f), jnp.float32),
            pltpu.VMEM((d, f), jnp.bfloat16),
            pltpu.VMEM((m, f), jnp.float32),
            pltpu.VMEM((MQ, f), jnp.bfloat16),
            pltpu.VMEM((m, d), jnp.bfloat16),
            pltpu.VMEM((m, d), jnp.bfloat16),
            pltpu.SemaphoreType.DMA((NC,)),
            pltpu.SemaphoreType.DMA((4,)),
            pltpu.SemaphoreType.DMA((2 * NC,)),
            pltpu.SemaphoreType.DMA((2 * NC,)),
            pltpu.SemaphoreType.DMA((2 * NC,)),
            pltpu.SemaphoreType.DMA((2 * NC,)),
        ],
        compiler_params=pltpu.CompilerParams(
            collective_id=0,
            vmem_limit_bytes=100 * 1024 * 1024,
        ),
    )(dy, W)


# device time: 29997 ns/iter; 1.6513x vs baseline; 1.0202x over previous
import jax
import jax.numpy as jnp
from jax import lax
from jax.experimental import pallas as pl
from jax.experimental.pallas import tpu as pltpu

MQ = 256
DC = 256
NC = 4


def kernel(dy, W):
    m, f = dy.shape
    d = W.shape[0]

    def body(
        dy_hbm,
        w_hbm,
        out_hbm,
        w_f32,
        out_vmem,
        w_bf,
        dy_f32,
        dy_bf,
        send_bf,
        recv_bf,
        w_sems,
        dy_sems,
        out_sems,
        x_send_sems,
        x_recv_sems,
        fwd_send_sems,
        fwd_recv_sems,
    ):
        my_x = lax.axis_index("x")
        my_y = lax.axis_index("y")
        my_z = lax.axis_index("z")
        py = lax.rem(my_y, 2)
        pz = lax.rem(my_z, 2)
        x_peer = (1 - my_x, my_y, my_z)
        y_partner = (my_x, my_y + 1 - 2 * py, my_z)
        z_partner = (my_x, my_y, my_z + 1 - 2 * pz)

        jA = 2 * py + pz
        jD = 3 - jA
        order = (jA, jD, 2 * (1 - py) + pz, 2 * py + (1 - pz))

        def q(j):
            return pl.ds(j * MQ, MQ)

        def col(c):
            return pl.ds(c * DC, DC)

        barrier = pltpu.get_barrier_semaphore()
        for nbr in (x_peer, y_partner, z_partner):
            pl.semaphore_signal(
                barrier, inc=1, device_id=nbr,
                device_id_type=pl.DeviceIdType.MESH,
            )

        w_cps = [
            pltpu.make_async_copy(
                w_hbm.at[col(c), :], w_f32.at[col(c), :], w_sems.at[c]
            )
            for c in range(NC)
        ]
        dy_cps = [
            pltpu.make_async_copy(
                dy_hbm.at[q(j), :], dy_f32.at[q(j), :], dy_sems.at[t]
            )
            for t, j in enumerate(order)
        ]
        w_cps[0].start()
        dy_cps[0].start()
        for c in range(1, NC):
            w_cps[c].start()
        for t in range(1, 4):
            dy_cps[t].start()

        def x_piece(t, j, c):
            return pltpu.make_async_remote_copy(
                src_ref=send_bf.at[q(j), col(c)],
                dst_ref=recv_bf.at[q(j), col(c)],
                send_sem=x_send_sems.at[t * NC + c],
                recv_sem=x_recv_sems.at[t * NC + c],
                device_id=x_peer,
                device_id_type=pl.DeviceIdType.MESH,
            )

        def fwd_piece(i, partner, c):
            return pltpu.make_async_remote_copy(
                src_ref=recv_bf.at[q(jA), col(c)],
                dst_ref=recv_bf.at[q(jA), col(c)],
                send_sem=fwd_send_sems.at[i * NC + c],
                recv_sem=fwd_recv_sems.at[i * NC + c],
                device_id=partner,
                device_id_type=pl.DeviceIdType.MESH,
            )

        for t, j in enumerate(order):
            dy_cps[t].wait()
            dy_bf[...] = dy_f32[q(j), :].astype(jnp.bfloat16)
            for c in range(NC):
                if t == 0:
                    w_cps[c].wait()
                    w_bf[col(c), :] = w_f32[col(c), :].astype(jnp.bfloat16)
                piece = lax.dot_general(
                    dy_bf[...],
                    w_bf[col(c), :],
                    dimension_numbers=(((1,), (1,)), ((), ())),
                    preferred_element_type=jnp.float32,
                )
                out_vmem[q(j), col(c)] = piece
                if t < 2:
                    send_bf[q(j), col(c)] = piece.astype(jnp.bfloat16)
                    if t == 0 and c == 0:
                        pl.semaphore_wait(barrier, 3)
                    x_piece(t, j, c).start()
                if t == 1:
                    x_piece(0, jA, c).wait_recv()
                    for i, partner in enumerate((y_partner, z_partner)):
                        fwd_piece(i, partner, c).start()

        def out_cp(k, j):
            return pltpu.make_async_copy(
                out_vmem.at[q(j), :], out_hbm.at[q(j), :], out_sems.at[k]
            )

        def finish_quarter(k, j):
            out_vmem[q(j), :] = out_vmem[q(j), :] + recv_bf[q(j), :].astype(
                jnp.float32
            )
            out_cp(k, j).start()

        finish_quarter(0, jA)
        for i, (jP, partner) in enumerate(
            ((order[2], y_partner), (order[3], z_partner))
        ):
            for c in range(NC):
                fwd_piece(i, partner, c).wait_recv()
            finish_quarter(1 + i, jP)
        for c in range(NC):
            x_piece(1, jD, c).wait_recv()
        finish_quarter(3, jD)

        for k, j in enumerate((jA, order[2], order[3], jD)):
            out_cp(k, j).wait()
        for t in range(2):
            for c in range(NC):
                x_piece(t, order[t], c).wait_send()
        for i, partner in enumerate((y_partner, z_partner)):
            for c in range(NC):
                fwd_piece(i, partner, c).wait_send()

    return pl.pallas_call(
        body,
        out_shape=jax.ShapeDtypeStruct((m, d), jnp.float32),
        in_specs=[
            pl.BlockSpec(memory_space=pl.ANY),
            pl.BlockSpec(memory_space=pl.ANY),
        ],
        out_specs=pl.BlockSpec(memory_space=pl.ANY),
        scratch_shapes=[
            pltpu.VMEM((d, f), jnp.float32),
            pltpu.VMEM((m, d), jnp.float32),
            pltpu.VMEM((d, f), jnp.bfloat16),
            pltpu.VMEM((m, f), jnp.float32),
            pltpu.VMEM((MQ, f), jnp.bfloat16),
            pltpu.VMEM((m, d), jnp.bfloat16),
            pltpu.VMEM((m, d), jnp.bfloat16),
            pltpu.SemaphoreType.DMA((NC,)),
            pltpu.SemaphoreType.DMA((4,)),
            pltpu.SemaphoreType.DMA((4,)),
            pltpu.SemaphoreType.DMA((2 * NC,)),
            pltpu.SemaphoreType.DMA((2 * NC,)),
            pltpu.SemaphoreType.DMA((2 * NC,)),
            pltpu.SemaphoreType.DMA((2 * NC,)),
        ],
        compiler_params=pltpu.CompilerParams(
            collective_id=0,
            vmem_limit_bytes=100 * 1024 * 1024,
        ),
    )(dy, W)


# device time: 28840 ns/iter; 1.7176x vs baseline; 1.0401x over previous
import jax
import jax.numpy as jnp
from jax import lax
from jax.experimental import pallas as pl
from jax.experimental.pallas import tpu as pltpu

MQ = 256
DC = 256
NC = 4


def kernel(dy, W):
    m, f = dy.shape
    d = W.shape[0]

    def body(
        dy_hbm,
        w_hbm,
        out_hbm,
        w_f32,
        out_vmem,
        w_bf,
        dy_f32,
        dy_bf,
        send_bf,
        recv_bf,
        recv_loc,
        w_sems,
        dy_sems,
        out_sems,
        x_send_sems,
        x_recv_sems,
        fwd_send_sems,
        fwd_recv_sems,
        loc_send_sems,
        loc_recv_sems,
    ):
        my_x = lax.axis_index("x")
        my_y = lax.axis_index("y")
        my_z = lax.axis_index("z")
        py = lax.rem(my_y, 2)
        pz = lax.rem(my_z, 2)
        x_peer = (1 - my_x, my_y, my_z)
        y_partner = (my_x, my_y + 1 - 2 * py, my_z)
        z_partner = (my_x, my_y, my_z + 1 - 2 * pz)

        jA = 2 * py + pz
        jD = 3 - jA
        order = (jA, jD, 2 * (1 - py) + pz, 2 * py + (1 - pz))

        def q(j):
            return pl.ds(j * MQ, MQ)

        def col(c):
            return pl.ds(c * DC, DC)

        barrier = pltpu.get_barrier_semaphore()
        for nbr in (x_peer, y_partner, z_partner):
            pl.semaphore_signal(
                barrier, inc=1, device_id=nbr,
                device_id_type=pl.DeviceIdType.MESH,
            )

        w_cps = [
            pltpu.make_async_copy(
                w_hbm.at[col(c), :], w_f32.at[col(c), :], w_sems.at[c]
            )
            for c in range(NC)
        ]
        dy_cps = [
            pltpu.make_async_copy(
                dy_hbm.at[q(j), :], dy_f32.at[q(j), :], dy_sems.at[t]
            )
            for t, j in enumerate((jA, jD))
        ]
        w_cps[0].start()
        dy_cps[0].start()
        for c in range(1, NC):
            w_cps[c].start()
        dy_cps[1].start()

        def x_piece(t, j, c):
            return pltpu.make_async_remote_copy(
                src_ref=send_bf.at[q(j), col(c)],
                dst_ref=recv_bf.at[q(j), col(c)],
                send_sem=x_send_sems.at[t * NC + c],
                recv_sem=x_recv_sems.at[t * NC + c],
                device_id=x_peer,
                device_id_type=pl.DeviceIdType.MESH,
            )

        def fwd_piece(i, partner, c):
            return pltpu.make_async_remote_copy(
                src_ref=recv_bf.at[q(jA), col(c)],
                dst_ref=recv_bf.at[q(jA), col(c)],
                send_sem=fwd_send_sems.at[i * NC + c],
                recv_sem=fwd_recv_sems.at[i * NC + c],
                device_id=partner,
                device_id_type=pl.DeviceIdType.MESH,
            )

        def loc_piece(i, partner, c):
            return pltpu.make_async_remote_copy(
                src_ref=send_bf.at[q(jA), col(c)],
                dst_ref=recv_loc.at[q(jA), col(c)],
                send_sem=loc_send_sems.at[i * NC + c],
                recv_sem=loc_recv_sems.at[i * NC + c],
                device_id=partner,
                device_id_type=pl.DeviceIdType.MESH,
            )

        for t, j in enumerate((jA, jD)):
            dy_cps[t].wait()
            dy_bf[...] = dy_f32[q(j), :].astype(jnp.bfloat16)
            for c in range(NC):
                if t == 0:
                    w_cps[c].wait()
                    w_bf[col(c), :] = w_f32[col(c), :].astype(jnp.bfloat16)
                piece = lax.dot_general(
                    dy_bf[...],
                    w_bf[col(c), :],
                    dimension_numbers=(((1,), (1,)), ((), ())),
                    preferred_element_type=jnp.float32,
                )
                out_vmem[q(j), col(c)] = piece
                send_bf[q(j), col(c)] = piece.astype(jnp.bfloat16)
                if t == 0 and c == 0:
                    pl.semaphore_wait(barrier, 3)
                x_piece(t, j, c).start()
                if t == 0:
                    for i, partner in enumerate((y_partner, z_partner)):
                        loc_piece(i, partner, c).start()
                if t == 1:
                    x_piece(0, jA, c).wait_recv()
                    for i, partner in enumerate((y_partner, z_partner)):
                        fwd_piece(i, partner, c).start()

        def out_cp(k, j):
            return pltpu.make_async_copy(
                out_vmem.at[q(j), :], out_hbm.at[q(j), :], out_sems.at[k]
            )

        def finish_quarter(k, j):
            out_vmem[q(j), :] = out_vmem[q(j), :] + recv_bf[q(j), :].astype(
                jnp.float32
            )
            out_cp(k, j).start()

        finish_quarter(0, jA)
        for c in range(NC):
            x_piece(1, jD, c).wait_recv()
        finish_quarter(1, jD)
        for i, (jP, partner) in enumerate(
            ((order[2], y_partner), (order[3], z_partner))
        ):
            for c in range(NC):
                loc_piece(i, partner, c).wait_recv()
                fwd_piece(i, partner, c).wait_recv()
            out_vmem[q(jP), :] = recv_loc[q(jP), :].astype(
                jnp.float32
            ) + recv_bf[q(jP), :].astype(jnp.float32)
            out_cp(2 + i, jP).start()

        for k, j in enumerate((jA, jD, order[2], order[3])):
            out_cp(k, j).wait()
        for t, j in enumerate((jA, jD)):
            for c in range(NC):
                x_piece(t, j, c).wait_send()
        for i, partner in enumerate((y_partner, z_partner)):
            for c in range(NC):
                fwd_piece(i, partner, c).wait_send()
                loc_piece(i, partner, c).wait_send()

    return pl.pallas_call(
        body,
        out_shape=jax.ShapeDtypeStruct((m, d), jnp.float32),
        in_specs=[
            pl.BlockSpec(memory_space=pl.ANY),
            pl.BlockSpec(memory_space=pl.ANY),
        ],
        out_specs=pl.BlockSpec(memory_space=pl.ANY),
        scratch_shapes=[
            pltpu.VMEM((d, f), jnp.float32),
            pltpu.VMEM((m, d), jnp.float32),
            pltpu.VMEM((d, f), jnp.bfloat16),
            pltpu.VMEM((m, f), jnp.float32),
            pltpu.VMEM((MQ, f), jnp.bfloat16),
            pltpu.VMEM((m, d), jnp.bfloat16),
            pltpu.VMEM((m, d), jnp.bfloat16),
            pltpu.VMEM((m, d), jnp.bfloat16),
            pltpu.SemaphoreType.DMA((NC,)),
            pltpu.SemaphoreType.DMA((2,)),
            pltpu.SemaphoreType.DMA((4,)),
            pltpu.SemaphoreType.DMA((2 * NC,)),
            pltpu.SemaphoreType.DMA((2 * NC,)),
            pltpu.SemaphoreType.DMA((2 * NC,)),
            pltpu.SemaphoreType.DMA((2 * NC,)),
            pltpu.SemaphoreType.DMA((2 * NC,)),
            pltpu.SemaphoreType.DMA((2 * NC,)),
        ],
        compiler_params=pltpu.CompilerParams(
            collective_id=0,
            vmem_limit_bytes=100 * 1024 * 1024,
        ),
    )(dy, W)
